# Initial kernel scaffold; baseline (speedup 1.0000x reference)
import functools

import jax
import jax.numpy as jnp
from jax import lax
from jax.experimental import pallas as pl
from jax.experimental.pallas import tpu as pltpu
from jax.experimental.pallas import tpu_sc as plsc

N = 10000
E = 320000
D = 128
H = 4
OUT = 32
HD = H * OUT

NC = 2
NS = 16
NW = NC * NS
CHUNK = 128
NCHUNKS = E // CHUNK
KMAX = (NCHUNKS + NW - 1) // NW
ROWS_PER_TILE = N // NS
DEN_W = 16
INV_SQRT_OUT = 1.0 / (OUT ** 0.5)


def _edge_body(ft_hbm, src_hbm, dst_hbm, s_out, den_out,
               src_v, dst_v, srows, drows, ebuf, s_sh, den_sh, sem_s, sem_d):
    cid = lax.axis_index("c")
    sid = lax.axis_index("s")
    w = sid * NC + cid

    z16 = jnp.zeros((16,), jnp.float32)

    def _zero_bufs(i, carry):
        for k in range(HD // 16):
            srows[i, pl.ds(k * 16, 16)] = z16
        ebuf[i, pl.ds(0, 16)] = z16
        return carry
    lax.fori_loop(0, CHUNK, _zero_bufs, 0)

    rows0 = sid * ROWS_PER_TILE
    for j in range(5):
        r = rows0 + j * 125
        pltpu.sync_copy(srows.at[pl.ds(0, 125)], s_sh.at[pl.ds(r, 125)])
        pltpu.sync_copy(ebuf.at[pl.ds(0, 125)], den_sh.at[pl.ds(r, 125)])
    plsc.subcore_barrier()

    def _chunk_body(k, carry):
        chunk = w + NW * k

        @pl.when(chunk < NCHUNKS)
        def _():
            base = chunk * CHUNK
            pltpu.sync_copy(src_hbm.at[pl.ds(base, CHUNK)], src_v)
            pltpu.sync_copy(dst_hbm.at[pl.ds(base, CHUNK)], dst_v)
            cs = pltpu.async_copy(ft_hbm.at[src_v], srows, sem_s)
            cd = pltpu.async_copy(ft_hbm.at[dst_v], drows, sem_d)
            cs.wait()
            cd.wait()

            def _edge(e, c2):
                for h in range(H):
                    s0 = srows[e, pl.ds(h * OUT, 16)]
                    s1 = srows[e, pl.ds(h * OUT + 16, 16)]
                    d0 = drows[e, pl.ds(h * OUT, 16)]
                    d1 = drows[e, pl.ds(h * OUT + 16, 16)]
                    prod = s0 * d0 + s1 * d1
                    tot = jnp.sum(prod) * INV_SQRT_OUT
                    wv = jnp.exp(jnp.full((16,), tot, jnp.float32))
                    ebuf[e, h] = wv[0]
                    srows[e, pl.ds(h * OUT, 16)] = s0 * wv
                    srows[e, pl.ds(h * OUT + 16, 16)] = s1 * wv
                return c2
            lax.fori_loop(0, CHUNK, _edge, 0)

            pltpu.sync_copy(srows, s_sh.at[dst_v], add=True)
            pltpu.sync_copy(ebuf, den_sh.at[dst_v], add=True)
        return carry
    lax.fori_loop(0, KMAX, _chunk_body, 0)

    plsc.subcore_barrier()
    for j in range(5):
        r = rows0 + j * 125
        pltpu.sync_copy(s_sh.at[pl.ds(r, 125)], s_out.at[cid, pl.ds(r, 125)])
        pltpu.sync_copy(den_sh.at[pl.ds(r, 125)], den_out.at[cid, pl.ds(r, 125)])


_edge_call = pl.kernel(
    _edge_body,
    out_type=(jax.ShapeDtypeStruct((NC, N, HD), jnp.float32),
              jax.ShapeDtypeStruct((NC, N, DEN_W), jnp.float32)),
    mesh=plsc.VectorSubcoreMesh(core_axis_name="c", subcore_axis_name="s"),
    scratch_types=[
        pltpu.VMEM((CHUNK,), jnp.int32),
        pltpu.VMEM((CHUNK,), jnp.int32),
        pltpu.VMEM((CHUNK, HD), jnp.float32),
        pltpu.VMEM((CHUNK, HD), jnp.float32),
        pltpu.VMEM((CHUNK, DEN_W), jnp.float32),
        pltpu.VMEM_SHARED((N, HD), jnp.float32),
        pltpu.VMEM_SHARED((N, DEN_W), jnp.float32),
    ] + [pltpu.SemaphoreType.DMA] * 2,
)

BLK = 2000


def _mm_body(x_ref, w_ref, o_ref):
    o_ref[...] = jnp.dot(x_ref[...], w_ref[...],
                         preferred_element_type=jnp.float32)


def _matmul(x, w):
    return pl.pallas_call(
        _mm_body,
        grid=(N // BLK,),
        in_specs=[pl.BlockSpec((BLK, D), lambda i: (i, 0)),
                  pl.BlockSpec((D, HD), lambda i: (0, 0))],
        out_specs=pl.BlockSpec((BLK, HD), lambda i: (i, 0)),
        out_shape=jax.ShapeDtypeStruct((N, HD), jnp.float32),
    )(x, w)


def _head_bcast_mat():
    col_head = lax.broadcasted_iota(jnp.int32, (DEN_W, HD), 1) // OUT
    row = lax.broadcasted_iota(jnp.int32, (DEN_W, HD), 0)
    return (col_head == row).astype(jnp.float32)


def _combine_mm_body(s_ref, den_ref, w_ref, o_ref):
    s = s_ref[0] + s_ref[1]
    den = den_ref[0] + den_ref[1]
    dmat = jnp.dot(den, _head_bcast_mat(), preferred_element_type=jnp.float32)
    h = s / jnp.maximum(dmat, 1e-9)
    h = jnp.maximum(h, 0.0)
    o_ref[...] = jnp.dot(h, w_ref[...], preferred_element_type=jnp.float32)


def _combine_mm(s, den, w):
    return pl.pallas_call(
        _combine_mm_body,
        grid=(N // BLK,),
        in_specs=[pl.BlockSpec((NC, BLK, HD), lambda i: (0, i, 0)),
                  pl.BlockSpec((NC, BLK, DEN_W), lambda i: (0, i, 0)),
                  pl.BlockSpec((D, HD), lambda i: (0, 0))],
        out_specs=pl.BlockSpec((BLK, HD), lambda i: (i, 0)),
        out_shape=jax.ShapeDtypeStruct((N, HD), jnp.float32),
    )(s, den, w)


def _combine_body(s_ref, den_ref, o_ref):
    s = s_ref[0] + s_ref[1]
    den = den_ref[0] + den_ref[1]
    dmat = jnp.dot(den, _head_bcast_mat(), preferred_element_type=jnp.float32)
    o_ref[...] = s / jnp.maximum(dmat, 1e-9)


def _combine(s, den):
    return pl.pallas_call(
        _combine_body,
        grid=(N // BLK,),
        in_specs=[pl.BlockSpec((NC, BLK, HD), lambda i: (0, i, 0)),
                  pl.BlockSpec((NC, BLK, DEN_W), lambda i: (0, i, 0))],
        out_specs=pl.BlockSpec((BLK, HD), lambda i: (i, 0)),
        out_shape=jax.ShapeDtypeStruct((N, HD), jnp.float32),
    )(s, den)


def kernel(features, edge_index, W1, W2):
    src = edge_index[0]
    dst = edge_index[1]
    ft1 = _matmul(features, W1)
    s1, d1 = _edge_call(ft1, src, dst)
    ft2 = _combine_mm(s1, d1, W2)
    s2, d2 = _edge_call(ft2, src, dst)
    return _combine(s2, d2)


# speedup vs baseline: 21.2314x; 21.2314x over previous
import jax
import jax.numpy as jnp
from jax import lax
from jax.experimental import pallas as pl
from jax.experimental.pallas import tpu as pltpu
from jax.experimental.pallas import tpu_sc as plsc

N = 10000
E = 320000
D = 128
H = 4
OUT = 32
HD = H * OUT
HC = 2
CW = HC * OUT

NC = 2
NS = 16
CHUNK = 80
EPT = E // NS
KMAX = EPT // CHUNK
ROWS_PER_TILE = 624
DEN_W = 16
INV_SQRT_OUT = 1.0 / (OUT ** 0.5)


def _edge_body(ft_hbm, src_hbm, dst_hbm, s_out, den_out,
               src_v, dst_v, dgat_v, srows, drows, ebuf, zrow, s_sh, den_sh,
               sem_s, sem_d):
    cid = lax.axis_index("c")
    sid = lax.axis_index("s")

    z16 = jnp.zeros((16,), jnp.float32)

    def _zero_bufs(i, carry):
        for k in range(CW // 16):
            zrow[i, pl.ds(k * 16, 16)] = z16
        ebuf[i, pl.ds(0, 16)] = z16
        return carry
    lax.fori_loop(0, 128, _zero_bufs, 0)

    base_r = pl.multiple_of(sid * ROWS_PER_TILE, 8)
    for o in (0, 128, 256, 384):
        pltpu.sync_copy(zrow, s_sh.at[pl.ds(base_r + o, 128)])
        pltpu.sync_copy(ebuf.at[pl.ds(0, 128)], den_sh.at[pl.ds(base_r + o, 128)])
    pltpu.sync_copy(zrow.at[pl.ds(0, 112)], s_sh.at[pl.ds(base_r + 512, 112)])
    pltpu.sync_copy(ebuf.at[pl.ds(0, 112)], den_sh.at[pl.ds(base_r + 512, 112)])

    @pl.when(sid == 0)
    def _():
        pltpu.sync_copy(zrow.at[pl.ds(0, 16)], s_sh.at[pl.ds(NS * ROWS_PER_TILE, 16)])
        pltpu.sync_copy(ebuf.at[pl.ds(0, 16)], den_sh.at[pl.ds(NS * ROWS_PER_TILE, 16)])
    plsc.subcore_barrier()

    row0 = cid * N
    ebase = pl.multiple_of(sid * EPT, 8)

    def _chunk_body(k, carry):
        base = ebase + k * CHUNK
        pltpu.sync_copy(src_hbm.at[pl.ds(base, CHUNK)], src_v)
        pltpu.sync_copy(dst_hbm.at[pl.ds(base, CHUNK)], dst_v)

        def _shift(i, c2):
            src_v[pl.ds(i * 16, 16)] = src_v[pl.ds(i * 16, 16)] + row0
            dgat_v[pl.ds(i * 16, 16)] = dst_v[pl.ds(i * 16, 16)] + row0
            return c2
        lax.fori_loop(0, CHUNK // 16, _shift, 0)

        cs = pltpu.async_copy(ft_hbm.at[src_v], srows, sem_s)
        cd = pltpu.async_copy(ft_hbm.at[dgat_v], drows, sem_d)
        cs.wait()
        cd.wait()

        lane = lax.iota(jnp.int32, 16)

        def _edge(e, c2):
            wvec = z16
            for h in range(HC):
                s0 = srows[e, pl.ds(h * OUT, 16)]
                s1 = srows[e, pl.ds(h * OUT + 16, 16)]
                d0 = drows[e, pl.ds(h * OUT, 16)]
                d1 = drows[e, pl.ds(h * OUT + 16, 16)]
                prod = s0 * d0 + s1 * d1
                tot = jnp.sum(prod) * INV_SQRT_OUT
                wv = jnp.exp(jnp.full((16,), tot, jnp.float32))
                wvec = jnp.where(lane == h, wv, wvec)
                srows[e, pl.ds(h * OUT, 16)] = s0 * wv
                srows[e, pl.ds(h * OUT + 16, 16)] = s1 * wv
            ebuf[e, pl.ds(0, 16)] = wvec
            return c2
        lax.fori_loop(0, CHUNK, _edge, 0)

        pltpu.sync_copy(srows, s_sh.at[dst_v], add=True)
        pltpu.sync_copy(ebuf.at[pl.ds(0, CHUNK)], den_sh.at[dst_v], add=True)
        return carry
    lax.fori_loop(0, KMAX, _chunk_body, 0)

    plsc.subcore_barrier()

    out_r0 = pl.multiple_of(cid * N + base_r, 8)
    for o, nr in ((0, 128), (128, 128), (256, 128), (384, 128), (512, 112)):
        pltpu.sync_copy(s_sh.at[pl.ds(base_r + o, nr)], zrow.at[pl.ds(0, nr)])
        pltpu.sync_copy(zrow.at[pl.ds(0, nr)], s_out.at[pl.ds(out_r0 + o, nr)])
        pltpu.sync_copy(den_sh.at[pl.ds(base_r + o, nr)], ebuf.at[pl.ds(0, nr)])
        pltpu.sync_copy(ebuf.at[pl.ds(0, nr)], den_out.at[pl.ds(out_r0 + o, nr)])

    @pl.when(sid == 0)
    def _():
        tail = NS * ROWS_PER_TILE
        tail_o = pl.multiple_of(cid * N + tail, 8)
        pltpu.sync_copy(s_sh.at[pl.ds(tail, 16)], zrow.at[pl.ds(0, 16)])
        pltpu.sync_copy(zrow.at[pl.ds(0, 16)], s_out.at[pl.ds(tail_o, 16)])
        pltpu.sync_copy(den_sh.at[pl.ds(tail, 16)], ebuf.at[pl.ds(0, 16)])
        pltpu.sync_copy(ebuf.at[pl.ds(0, 16)], den_out.at[pl.ds(tail_o, 16)])


_edge_call_flat = pl.kernel(
    _edge_body,
    out_type=(jax.ShapeDtypeStruct((NC * N, CW), jnp.float32),
              jax.ShapeDtypeStruct((NC * N, DEN_W), jnp.float32)),
    mesh=plsc.VectorSubcoreMesh(core_axis_name="c", subcore_axis_name="s"),
    compiler_params=pltpu.CompilerParams(needs_layout_passes=False,
                                         use_tc_tiling_on_sc=False),
    scratch_types=[
        pltpu.VMEM((CHUNK,), jnp.int32),
        pltpu.VMEM((CHUNK,), jnp.int32),
        pltpu.VMEM((CHUNK,), jnp.int32),
        pltpu.VMEM((CHUNK, CW), jnp.float32),
        pltpu.VMEM((CHUNK, CW), jnp.float32),
        pltpu.VMEM((128, DEN_W), jnp.float32),
        pltpu.VMEM((128, CW), jnp.float32),
        pltpu.VMEM_SHARED((N, CW), jnp.float32),
        pltpu.VMEM_SHARED((N, DEN_W), jnp.float32),
    ] + [pltpu.SemaphoreType.DMA] * 2,
)


BLK = 2000


def _mm_body(x_ref, w_ref, o_ref):
    y = jnp.dot(x_ref[...], w_ref[...], preferred_element_type=jnp.float32)
    o_ref[0] = y[:, :CW]
    o_ref[1] = y[:, CW:]


def _matmul_split(x, w):
    return pl.pallas_call(
        _mm_body,
        grid=(N // BLK,),
        in_specs=[pl.BlockSpec((BLK, D), lambda i: (i, 0)),
                  pl.BlockSpec((D, HD), lambda i: (0, 0))],
        out_specs=pl.BlockSpec((NC, BLK, CW), lambda i: (0, i, 0)),
        out_shape=jax.ShapeDtypeStruct((NC, N, CW), jnp.float32),
    )(x, w)


def _head_bcast_mat():
    col_head = lax.broadcasted_iota(jnp.int32, (DEN_W, CW), 1) // OUT
    row = lax.broadcasted_iota(jnp.int32, (DEN_W, CW), 0)
    return (col_head == row).astype(jnp.float32)


def _normalize(s_ref, den_ref):
    pmat = _head_bcast_mat()
    cols = []
    for c in range(NC):
        dmat = jnp.dot(den_ref[c], pmat, preferred_element_type=jnp.float32)
        cols.append(s_ref[c] / jnp.maximum(dmat, 1e-9))
    return jnp.concatenate(cols, axis=1)


def _combine_mm_body(s_ref, den_ref, w_ref, o_ref):
    h = jnp.maximum(_normalize(s_ref, den_ref), 0.0)
    y = jnp.dot(h, w_ref[...], preferred_element_type=jnp.float32)
    o_ref[0] = y[:, :CW]
    o_ref[1] = y[:, CW:]


def _combine_mm_split(s, den, w):
    return pl.pallas_call(
        _combine_mm_body,
        grid=(N // BLK,),
        in_specs=[pl.BlockSpec((NC, BLK, CW), lambda i: (0, i, 0)),
                  pl.BlockSpec((NC, BLK, DEN_W), lambda i: (0, i, 0)),
                  pl.BlockSpec((D, HD), lambda i: (0, 0))],
        out_specs=pl.BlockSpec((NC, BLK, CW), lambda i: (0, i, 0)),
        out_shape=jax.ShapeDtypeStruct((NC, N, CW), jnp.float32),
    )(s, den, w)


def _combine_body(s_ref, den_ref, o_ref):
    o_ref[...] = _normalize(s_ref, den_ref)


def _combine(s, den):
    return pl.pallas_call(
        _combine_body,
        grid=(N // BLK,),
        in_specs=[pl.BlockSpec((NC, BLK, CW), lambda i: (0, i, 0)),
                  pl.BlockSpec((NC, BLK, DEN_W), lambda i: (0, i, 0))],
        out_specs=pl.BlockSpec((BLK, HD), lambda i: (i, 0)),
        out_shape=jax.ShapeDtypeStruct((N, HD), jnp.float32),
    )(s, den)


def _edge_call(ft_split, src, dst):
    s, den = _edge_call_flat(ft_split.reshape(NC * N, CW), src, dst)
    return s.reshape(NC, N, CW), den.reshape(NC, N, DEN_W)


def kernel(features, edge_index, W1, W2):
    src = edge_index[0]
    dst = edge_index[1]
    ft1 = _matmul_split(features, W1)
    s1, d1 = _edge_call(ft1, src, dst)
    ft2 = _combine_mm_split(s1, d1, W2)
    s2, d2 = _edge_call(ft2, src, dst)
    return _combine(s2, d2)
